# Initial kernel scaffold; baseline (speedup 1.0000x reference)
import jax
import jax.numpy as jnp
from jax.experimental import pallas as pl


def kernel(pos, Z, edge_index, triplet_index, atom_emb, W_emb, Wo_rbf, Wo_dense, Wo_out, Wi_rbf, Wi_sbf, Wi_src, Wi_bil, Wi_before, Wi_after):
    raise NotImplementedError("write your pallas kernel here")


# speedup vs baseline: 1.0185x; 1.0185x over previous
import functools

import jax
import jax.numpy as jnp
from jax import lax
from jax.experimental import pallas as pl
from jax.experimental.pallas import tpu as pltpu

N = 10000
E = 160000
T = 320000
EMB = 128
NR = 6
NS = 7
NB = 4
NBLK = 2
NT = 12
CUTOFF = 5.0
P_ENV = 5

BE = 2000
BT = 2000
BN = 2000

_A = -(P_ENV + 1) * (P_ENV + 2) / 2.0
_B = P_ENV * (P_ENV + 2.0)
_C = -P_ENV * (P_ENV + 1) / 2.0


def _swish(x):
    return x * jax.nn.sigmoid(x)


def _edge_body(ps_ref, pd_ref, A1_ref, A2_ref, W3_ref, Wsbf_ref, Worbf0_ref,
               m_ref, rbf8_ref, geo_ref, vecp_ref, t0_ref):
    ps = ps_ref[...]
    pd = pd_ref[...]
    vec = pd[:, :3] - ps[:, :3]
    d2 = jnp.sum(vec * vec, axis=-1) + 1e-9
    d = jnp.sqrt(d2)
    u = d / CUTOFF
    u4 = (u * u) * (u * u)
    env = 1.0 / u + _A * u4 + _B * u4 * u + _C * u4 * u * u
    scale = env / d
    k48 = lax.broadcasted_iota(jnp.int32, (1, 48), 1).astype(jnp.float32) + 1.0
    rbf_env48 = scale[:, None] * jnp.sin(jnp.pi * k48 * u[:, None])
    i8 = lax.broadcasted_iota(jnp.int32, (1, 8), 1)
    rbf8 = jnp.where(i8 < NR, rbf_env48[:, :8], 0.0)
    c56 = jnp.dot(rbf_env48, Wsbf_ref[...], preferred_element_type=jnp.float32)
    geo = jnp.concatenate([vec, c56, jnp.zeros((vec.shape[0], 5), jnp.float32)], axis=1)
    zs = ps[:, 3].astype(jnp.int32)
    zd = pd[:, 3].astype(jnp.int32)
    ids = lax.broadcasted_iota(jnp.int32, (1, 96), 1)
    oh_s = (zs[:, None] == ids).astype(jnp.float32)
    oh_d = (zd[:, None] == ids).astype(jnp.float32)
    m_pre = (jnp.dot(oh_s, A1_ref[...], preferred_element_type=jnp.float32)
             + jnp.dot(oh_d, A2_ref[...], preferred_element_type=jnp.float32)
             + jnp.dot(rbf8, W3_ref[...], preferred_element_type=jnp.float32))
    m = _swish(m_pre)
    m_ref[...] = m
    rbf8_ref[...] = rbf8
    geo_ref[...] = geo
    vecp_ref[...] = jnp.concatenate(
        [vec, jnp.zeros((vec.shape[0], 13), jnp.float32)], axis=1)
    t0_ref[...] = m * jnp.dot(rbf8, Worbf0_ref[...],
                              preferred_element_type=jnp.float32)


def _edge_stage(ps, pd, A1, A2, W3p, Wsbf_bd, Worbf0):
    grid = (E // BE,)
    rowspec = lambda w: pl.BlockSpec((BE, w), lambda i: (i, 0))
    full = lambda a: pl.BlockSpec(a.shape, lambda i: (0,) * a.ndim)
    return pl.pallas_call(
        _edge_body,
        grid=grid,
        in_specs=[rowspec(16), rowspec(16), full(A1), full(A2), full(W3p),
                  full(Wsbf_bd), full(Worbf0)],
        out_specs=[rowspec(EMB), rowspec(8), rowspec(64), rowspec(16),
                   rowspec(EMB)],
        out_shape=[
            jax.ShapeDtypeStruct((E, EMB), jnp.float32),
            jax.ShapeDtypeStruct((E, 8), jnp.float32),
            jax.ShapeDtypeStruct((E, 64), jnp.float32),
            jax.ShapeDtypeStruct((E, 16), jnp.float32),
            jax.ShapeDtypeStruct((E, EMB), jnp.float32),
        ],
    )(ps, pd, A1, A2, W3p, Wsbf_bd, Worbf0)


def _tri_init_body(geo_ref, vtd_ref, sbf8_ref):
    geo = geo_ref[...]
    r1 = geo[:, :3]
    r2 = vtd_ref[...][:, :3]
    xdot = jnp.sum(r1 * r2, axis=-1)
    cr0 = r1[:, 1] * r2[:, 2] - r1[:, 2] * r2[:, 1]
    cr1 = r1[:, 2] * r2[:, 0] - r1[:, 0] * r2[:, 2]
    cr2 = r1[:, 0] * r2[:, 1] - r1[:, 1] * r2[:, 0]
    ycr = jnp.sqrt(cr0 * cr0 + cr1 * cr1 + cr2 * cr2 + 1e-9)
    angle = jnp.arctan2(ycr, xdot)
    cols = []
    for blk in range(NBLK):
        for b in range(NB):
            acc = jnp.zeros_like(angle)
            for l in range(NS):
                acc = acc + jnp.cos(l * angle) * geo[:, 3 + blk * 28 + l * NB + b]
            cols.append(acc)
    sbf8_ref[...] = jnp.stack(cols, axis=1)


def _tri_init_stage(geo_t, vec_td):
    rowspec = lambda w: pl.BlockSpec((BT, w), lambda i: (i, 0))
    return pl.pallas_call(
        _tri_init_body,
        grid=(T // BT,),
        in_specs=[rowspec(64), rowspec(16)],
        out_specs=rowspec(8),
        out_shape=jax.ShapeDtypeStruct((T, 8), jnp.float32),
    )(geo_t, vec_td)


def _xsrc_body(m_ref, W_ref, x_ref):
    x_ref[...] = _swish(jnp.dot(m_ref[...], W_ref[...],
                                preferred_element_type=jnp.float32))


def _xsrc_stage(m, W):
    rowspec = pl.BlockSpec((BE, EMB), lambda i: (i, 0))
    return pl.pallas_call(
        _xsrc_body,
        grid=(E // BE,),
        in_specs=[rowspec, pl.BlockSpec(W.shape, lambda i: (0, 0))],
        out_specs=rowspec,
        out_shape=jax.ShapeDtypeStruct((E, EMB), jnp.float32),
    )(m, W)


def _bilinear_body(blk, xk_ref, sbf8_ref, Bcat_ref, msg_ref):
    z = jnp.dot(xk_ref[...], Bcat_ref[...], preferred_element_type=jnp.float32)
    sbf8 = sbf8_ref[...]
    acc = jnp.zeros((z.shape[0], EMB), jnp.float32)
    for b in range(NB):
        w = sbf8[:, blk * NB + b:blk * NB + b + 1]
        acc = acc + z[:, b * EMB:(b + 1) * EMB] * w
    msg_ref[...] = acc


def _bilinear_stage(xk, sbf8, Bcat, blk):
    rowspec = lambda w: pl.BlockSpec((BT, w), lambda i: (i, 0))
    return pl.pallas_call(
        functools.partial(_bilinear_body, blk),
        grid=(T // BT,),
        in_specs=[rowspec(EMB), rowspec(8),
                  pl.BlockSpec(Bcat.shape, lambda i: (0, 0))],
        out_specs=rowspec(EMB),
        out_shape=jax.ShapeDtypeStruct((T, EMB), jnp.float32),
    )(xk, sbf8, Bcat)


def _post_body(m_ref, rbf8_ref, agg_ref, Wirbf_ref, Wb_ref, Wa0_ref, Wa1_ref,
               Worbf_ref, mout_ref, t_ref):
    m = m_ref[...]
    rbf8 = rbf8_ref[...]
    m_rbf = m * jnp.dot(rbf8, Wirbf_ref[...], preferred_element_type=jnp.float32)
    h2 = _swish(jnp.dot(m_rbf + agg_ref[...], Wb_ref[...],
                        preferred_element_type=jnp.float32))
    mm = m + h2
    mm = _swish(jnp.dot(mm, Wa0_ref[...], preferred_element_type=jnp.float32))
    mm = _swish(jnp.dot(mm, Wa1_ref[...], preferred_element_type=jnp.float32))
    mout_ref[...] = mm
    t_ref[...] = mm * jnp.dot(rbf8, Worbf_ref[...],
                              preferred_element_type=jnp.float32)


def _post_stage(m, rbf8, agg, Wirbf, Wb, Wa0, Wa1, Worbf):
    rowspec = lambda w: pl.BlockSpec((BE, w), lambda i: (i, 0))
    full = lambda a: pl.BlockSpec(a.shape, lambda i: (0, 0))
    return pl.pallas_call(
        _post_body,
        grid=(E // BE,),
        in_specs=[rowspec(EMB), rowspec(8), rowspec(EMB), full(Wirbf),
                  full(Wb), full(Wa0), full(Wa1), full(Worbf)],
        out_specs=[rowspec(EMB), rowspec(EMB)],
        out_shape=[jax.ShapeDtypeStruct((E, EMB), jnp.float32),
                   jax.ShapeDtypeStruct((E, EMB), jnp.float32)],
    )(m, rbf8, agg, Wirbf, Wb, Wa0, Wa1, Worbf)


def _node_body(node_ref, Wd0_ref, Wd1_ref, Wd2_ref, Wo_ref, p_ref):
    h = node_ref[...]
    h = _swish(jnp.dot(h, Wd0_ref[...], preferred_element_type=jnp.float32))
    h = _swish(jnp.dot(h, Wd1_ref[...], preferred_element_type=jnp.float32))
    h = _swish(jnp.dot(h, Wd2_ref[...], preferred_element_type=jnp.float32))
    p_ref[...] = jnp.dot(h, Wo_ref[...], preferred_element_type=jnp.float32)


def _node_stage(node, Wd, Wo):
    return pl.pallas_call(
        _node_body,
        grid=(N // BN,),
        in_specs=[pl.BlockSpec((BN, EMB), lambda i: (i, 0))]
        + [pl.BlockSpec((EMB, EMB), lambda i: (0, 0))] * 3
        + [pl.BlockSpec(Wo.shape, lambda i: (0, 0))],
        out_specs=pl.BlockSpec((BN, NT), lambda i: (i, 0)),
        out_shape=jax.ShapeDtypeStruct((N, NT), jnp.float32),
    )(node, Wd[0], Wd[1], Wd[2], Wo)


def kernel(pos, Z, edge_index, triplet_index, atom_emb, W_emb, Wo_rbf,
           Wo_dense, Wo_out, Wi_rbf, Wi_sbf, Wi_src, Wi_bil, Wi_before,
           Wi_after):
    f32 = jnp.float32
    src = edge_index[0]
    dst = edge_index[1]
    ts = triplet_index[0]
    td = triplet_index[1]

    A1 = jnp.pad(atom_emb, ((0, 1), (0, 0))) @ W_emb[:EMB]
    A2 = jnp.pad(atom_emb, ((0, 1), (0, 0))) @ W_emb[EMB:2 * EMB]
    W3p = jnp.pad(W_emb[2 * EMB:], ((0, 2), (0, 0)))
    Wsbf_bd = jnp.zeros((48, 56), f32)
    for blk in range(NBLK):
        for l in range(NS):
            Wsbf_bd = Wsbf_bd.at[l * NR:(l + 1) * NR,
                                 blk * 28 + l * NB:blk * 28 + (l + 1) * NB].set(
                Wi_sbf[blk, l * NR:(l + 1) * NR, :])
    Wo_rbf8 = jnp.pad(Wo_rbf, ((0, 0), (0, 2), (0, 0)))
    Wi_rbf8 = jnp.pad(Wi_rbf, ((0, 0), (0, 2), (0, 0)))

    pos_z = jnp.concatenate(
        [pos, Z.astype(f32)[:, None], jnp.zeros((N, 12), f32)], axis=1)

    ps = pos_z[src]
    pd = pos_z[dst]

    m, rbf8, geo, vecp, t0 = _edge_stage(ps, pd, A1, A2, W3p, Wsbf_bd,
                                         Wo_rbf8[0])

    geo_t = geo[ts]
    vec_td = vecp[td]
    sbf8 = _tri_init_stage(geo_t, vec_td)

    node0 = jax.ops.segment_sum(t0, dst, num_segments=N)
    P = _node_stage(node0, Wo_dense[0], Wo_out[0])

    for i in range(NBLK):
        x = _xsrc_stage(m, Wi_src[i])
        xk = x[ts]
        Bcat = Wi_bil[i].reshape(EMB, NB * EMB)
        msg = _bilinear_stage(xk, sbf8, Bcat, i)
        agg = jax.ops.segment_sum(msg, td, num_segments=E)
        m, t = _post_stage(m, rbf8, agg, Wi_rbf8[i], Wi_before[i],
                           Wi_after[i, 0], Wi_after[i, 1], Wo_rbf8[i + 1])
        node = jax.ops.segment_sum(t, dst, num_segments=N)
        P = P + _node_stage(node, Wo_dense[i + 1], Wo_out[i + 1])

    return P


# speedup vs baseline: 1.3118x; 1.2880x over previous
import functools

import jax
import jax.numpy as jnp
from jax import lax
from jax.experimental import pallas as pl
from jax.experimental.pallas import tpu as pltpu
from jax.experimental.pallas import tpu_sc as plsc

N = 10000
E = 160000
T = 320000
EMB = 128
NR = 6
NS = 7
NB = 4
NBLK = 2
NT = 12
CUTOFF = 5.0
P_ENV = 5

BE = 2000
BT = 2000
BN = 2000

_A = -(P_ENV + 1) * (P_ENV + 2) / 2.0
_B = P_ENV * (P_ENV + 2.0)
_C = -P_ENV * (P_ENV + 1) / 2.0


def _swish(x):
    return x * jax.nn.sigmoid(x)


def _edge_body(ps_ref, pd_ref, A1_ref, A2_ref, W3_ref, Wsbf_ref, Worbf0_ref,
               m_ref, rbf8_ref, geo_ref, t0_ref):
    ps = ps_ref[...][:, :16]
    pd = pd_ref[...][:, :16]
    vec = pd[:, :3] - ps[:, :3]
    d2 = jnp.sum(vec * vec, axis=-1) + 1e-9
    d = jnp.sqrt(d2)
    u = d / CUTOFF
    u4 = (u * u) * (u * u)
    env = 1.0 / u + _A * u4 + _B * u4 * u + _C * u4 * u * u
    scale = env / d
    k48 = lax.broadcasted_iota(jnp.int32, (1, 48), 1).astype(jnp.float32) + 1.0
    rbf_env48 = scale[:, None] * jnp.sin(jnp.pi * k48 * u[:, None])
    i8 = lax.broadcasted_iota(jnp.int32, (1, 8), 1)
    rbf8 = jnp.where(i8 < NR, rbf_env48[:, :8], 0.0)
    c56 = jnp.dot(rbf_env48, Wsbf_ref[...], preferred_element_type=jnp.float32)
    geo = jnp.concatenate(
        [vec, jnp.zeros((vec.shape[0], 1), jnp.float32), c56,
         jnp.zeros((vec.shape[0], 68), jnp.float32)], axis=1)
    zs = ps[:, 3].astype(jnp.int32)
    zd = pd[:, 3].astype(jnp.int32)
    ids = lax.broadcasted_iota(jnp.int32, (1, 96), 1)
    oh_s = (zs[:, None] == ids).astype(jnp.float32)
    oh_d = (zd[:, None] == ids).astype(jnp.float32)
    m_pre = (jnp.dot(oh_s, A1_ref[...], preferred_element_type=jnp.float32)
             + jnp.dot(oh_d, A2_ref[...], preferred_element_type=jnp.float32)
             + jnp.dot(rbf8, W3_ref[...], preferred_element_type=jnp.float32))
    m = _swish(m_pre)
    m_ref[...] = m
    rbf8_ref[...] = rbf8
    geo_ref[...] = geo
    t0_ref[...] = m * jnp.dot(rbf8, Worbf0_ref[...],
                              preferred_element_type=jnp.float32)


def _edge_stage(ps, pd, A1, A2, W3p, Wsbf_bd, Worbf0):
    grid = (E // BE,)
    rowspec = lambda w: pl.BlockSpec((BE, w), lambda i: (i, 0))
    full = lambda a: pl.BlockSpec(a.shape, lambda i: (0,) * a.ndim)
    return pl.pallas_call(
        _edge_body,
        grid=grid,
        in_specs=[rowspec(EMB), rowspec(EMB), full(A1), full(A2), full(W3p),
                  full(Wsbf_bd), full(Worbf0)],
        out_specs=[rowspec(EMB), rowspec(8), rowspec(EMB), rowspec(EMB)],
        out_shape=[
            jax.ShapeDtypeStruct((E, EMB), jnp.float32),
            jax.ShapeDtypeStruct((E, 8), jnp.float32),
            jax.ShapeDtypeStruct((E, EMB), jnp.float32),
            jax.ShapeDtypeStruct((E, EMB), jnp.float32),
        ],
    )(ps, pd, A1, A2, W3p, Wsbf_bd, Worbf0)


def _tri_init_body(geo_ref, vtd_ref, sbf8_ref):
    geo = geo_ref[...]
    r1 = geo[:, :3]
    r2 = vtd_ref[...][:, :3]
    xdot = jnp.sum(r1 * r2, axis=-1)
    cr0 = r1[:, 1] * r2[:, 2] - r1[:, 2] * r2[:, 1]
    cr1 = r1[:, 2] * r2[:, 0] - r1[:, 0] * r2[:, 2]
    cr2 = r1[:, 0] * r2[:, 1] - r1[:, 1] * r2[:, 0]
    ycr = jnp.sqrt(cr0 * cr0 + cr1 * cr1 + cr2 * cr2 + 1e-9)
    angle = jnp.arctan2(ycr, xdot)
    cols = []
    for blk in range(NBLK):
        for b in range(NB):
            acc = jnp.zeros_like(angle)
            for l in range(NS):
                acc = acc + jnp.cos(l * angle) * geo[:, 4 + blk * 28 + l * NB + b]
            cols.append(acc)
    sbf8_ref[...] = jnp.stack(cols, axis=1)


def _tri_init_stage(geo_t, vec_td):
    rowspec = lambda w: pl.BlockSpec((BT, w), lambda i: (i, 0))
    return pl.pallas_call(
        _tri_init_body,
        grid=(T // BT,),
        in_specs=[rowspec(EMB), rowspec(EMB)],
        out_specs=rowspec(8),
        out_shape=jax.ShapeDtypeStruct((T, 8), jnp.float32),
    )(geo_t, vec_td)


def _xsrc_body(m_ref, W_ref, x_ref):
    x_ref[...] = _swish(jnp.dot(m_ref[...], W_ref[...],
                                preferred_element_type=jnp.float32))


def _xsrc_stage(m, W):
    rowspec = pl.BlockSpec((BE, EMB), lambda i: (i, 0))
    return pl.pallas_call(
        _xsrc_body,
        grid=(E // BE,),
        in_specs=[rowspec, pl.BlockSpec(W.shape, lambda i: (0, 0))],
        out_specs=rowspec,
        out_shape=jax.ShapeDtypeStruct((E, EMB), jnp.float32),
    )(m, W)


def _bilinear_body(blk, xk_ref, sbf8_ref, Bcat_ref, msg_ref):
    z = jnp.dot(xk_ref[...].astype(jnp.bfloat16), Bcat_ref[...],
                preferred_element_type=jnp.float32)
    sbf8 = sbf8_ref[...]
    acc = jnp.zeros((z.shape[0], EMB), jnp.float32)
    for b in range(NB):
        w = sbf8[:, blk * NB + b:blk * NB + b + 1]
        acc = acc + z[:, b * EMB:(b + 1) * EMB] * w
    msg_ref[...] = acc


def _bilinear_stage(xk, sbf8, Bcat, blk):
    rowspec = lambda w: pl.BlockSpec((BT, w), lambda i: (i, 0))
    return pl.pallas_call(
        functools.partial(_bilinear_body, blk),
        grid=(T // BT,),
        in_specs=[rowspec(EMB), rowspec(8),
                  pl.BlockSpec(Bcat.shape, lambda i: (0, 0))],
        out_specs=rowspec(EMB),
        out_shape=jax.ShapeDtypeStruct((T, EMB), jnp.float32),
    )(xk, sbf8, Bcat)


def _post_body(m_ref, rbf8_ref, agg_ref, Wirbf_ref, Wb_ref, Wa0_ref, Wa1_ref,
               Worbf_ref, mout_ref, t_ref):
    m = m_ref[...]
    rbf8 = rbf8_ref[...]
    m_rbf = m * jnp.dot(rbf8, Wirbf_ref[...], preferred_element_type=jnp.float32)
    h2 = _swish(jnp.dot(m_rbf + agg_ref[...], Wb_ref[...],
                        preferred_element_type=jnp.float32))
    mm = m + h2
    mm = _swish(jnp.dot(mm, Wa0_ref[...], preferred_element_type=jnp.float32))
    mm = _swish(jnp.dot(mm, Wa1_ref[...], preferred_element_type=jnp.float32))
    mout_ref[...] = mm
    t_ref[...] = mm * jnp.dot(rbf8, Worbf_ref[...],
                              preferred_element_type=jnp.float32)


def _post_stage(m, rbf8, agg, Wirbf, Wb, Wa0, Wa1, Worbf):
    rowspec = lambda w: pl.BlockSpec((BE, w), lambda i: (i, 0))
    full = lambda a: pl.BlockSpec(a.shape, lambda i: (0, 0))
    return pl.pallas_call(
        _post_body,
        grid=(E // BE,),
        in_specs=[rowspec(EMB), rowspec(8), rowspec(EMB), full(Wirbf),
                  full(Wb), full(Wa0), full(Wa1), full(Worbf)],
        out_specs=[rowspec(EMB), rowspec(EMB)],
        out_shape=[jax.ShapeDtypeStruct((E, EMB), jnp.float32),
                   jax.ShapeDtypeStruct((E, EMB), jnp.float32)],
    )(m, rbf8, agg, Wirbf, Wb, Wa0, Wa1, Worbf)


def _node_body(node_ref, Wd0_ref, Wd1_ref, Wd2_ref, Wo_ref, p_ref):
    h = node_ref[0] + node_ref[1]
    h = _swish(jnp.dot(h, Wd0_ref[...], preferred_element_type=jnp.float32))
    h = _swish(jnp.dot(h, Wd1_ref[...], preferred_element_type=jnp.float32))
    h = _swish(jnp.dot(h, Wd2_ref[...], preferred_element_type=jnp.float32))
    p_ref[...] = jnp.dot(h, Wo_ref[...], preferred_element_type=jnp.float32)


def _node_stage(node, Wd, Wo):
    return pl.pallas_call(
        _node_body,
        grid=(N // BN,),
        in_specs=[pl.BlockSpec((2, BN, EMB), lambda i: (0, i, 0))]
        + [pl.BlockSpec((EMB, EMB), lambda i: (0, 0))] * 3
        + [pl.BlockSpec(Wo.shape, lambda i: (0, 0))],
        out_specs=pl.BlockSpec((BN, NT), lambda i: (i, 0)),
        out_shape=jax.ShapeDtypeStruct((N, NT), jnp.float32),
    )(node, Wd[0], Wd[1], Wd[2], Wo)


NW = 32


@functools.partial(jax.jit, static_argnums=(2,))
def _sc_gather(table, idx, ch=80):
    V, D = table.shape
    B = idx.shape[0]
    bpw = B // NW
    nch = bpw // ch
    mesh = plsc.VectorSubcoreMesh(core_axis_name="c", subcore_axis_name="s")

    @functools.partial(
        pl.kernel, mesh=mesh,
        out_type=jax.ShapeDtypeStruct((B, D), jnp.float32),
        scratch_types=[
            pltpu.VMEM((bpw,), jnp.int32),
            pltpu.VMEM((2, ch, D), jnp.float32),
            pltpu.SemaphoreType.DMA,
            pltpu.SemaphoreType.DMA,
        ],
    )
    def k(table_hbm, idx_hbm, out_hbm, idx_v, rows_v, sem_g, sem_s):
        wid = lax.axis_index("s") * 2 + lax.axis_index("c")
        base = pl.multiple_of(wid * bpw, 8)
        pltpu.sync_copy(idx_hbm.at[pl.ds(base, bpw)], idx_v)

        def g_desc(c, b):
            return pltpu.make_async_copy(
                table_hbm.at[idx_v.at[pl.ds(pl.multiple_of(c * ch, 8), ch)]],
                rows_v.at[b], sem_g)

        def s_desc(c, b):
            return pltpu.make_async_copy(
                rows_v.at[b],
                out_hbm.at[pl.ds(pl.multiple_of(base + c * ch, 8), ch)], sem_s)

        g_desc(0, 0).start()

        def step(c, carry):
            b = lax.rem(c, 2)
            g_desc(c, b).wait()

            @pl.when(c >= 1)
            def _():
                s_desc(c - 1, 1 - b).wait()

            @pl.when(c + 1 < nch)
            def _():
                g_desc(c + 1, 1 - b).start()

            s_desc(c, b).start()
            return carry

        lax.fori_loop(0, nch, step, 0)
        s_desc(nch - 1, (nch - 1) % 2).wait()

    return k(table, idx)


@functools.partial(jax.jit, static_argnums=(3, 4))
def _sc_scatter_node(vals, idx3, zrows, nrows, ch):
    B, D = vals.shape
    nch = idx3.shape[2]
    mesh = plsc.VectorSubcoreMesh(core_axis_name="c", subcore_axis_name="s")

    @functools.partial(
        pl.kernel, mesh=mesh,
        out_type=jax.ShapeDtypeStruct((2, nrows, D), jnp.float32),
        scratch_types=[
            pltpu.VMEM((nch, ch), jnp.int32),
            pltpu.VMEM((ch, D), jnp.float32),
            pltpu.VMEM_SHARED((nrows, D), jnp.float32),
            pltpu.SemaphoreType.DMA,
        ],
    )
    def k(vals_hbm, idx_hbm, zero_hbm, out_hbm, idx_v, rows_v, acc_sh, sem):
        cid = lax.axis_index("c")
        sid = lax.axis_index("s")
        half = nrows // 2
        @pl.when(sid < 2)
        def _():
            pltpu.sync_copy(zero_hbm,
                            acc_sh.at[pl.ds(pl.multiple_of(sid * half, 8), half)])
        plsc.subcore_barrier()
        pltpu.sync_copy(idx_hbm.at[cid, sid], idx_v)
        row0 = (cid * 16 + sid) * (nch * ch)

        def step(c, carry):
            pltpu.sync_copy(
                vals_hbm.at[pl.ds(pl.multiple_of(row0 + c * ch, 8), ch)], rows_v)
            pltpu.sync_copy(rows_v, acc_sh.at[idx_v.at[c]], add=True)
            return carry

        lax.fori_loop(0, nch, step, 0)
        plsc.subcore_barrier()

        @pl.when(sid < 2)
        def _():
            off = pl.multiple_of(sid * half, 8)
            pltpu.sync_copy(acc_sh.at[pl.ds(off, half)],
                            out_hbm.at[cid, pl.ds(off, half)])

    return k(vals, idx3, zrows)


def kernel(pos, Z, edge_index, triplet_index, atom_emb, W_emb, Wo_rbf,
           Wo_dense, Wo_out, Wi_rbf, Wi_sbf, Wi_src, Wi_bil, Wi_before,
           Wi_after):
    f32 = jnp.float32
    src = edge_index[0]
    dst = edge_index[1]
    ts = triplet_index[0]
    td = triplet_index[1]

    A1 = jnp.pad(atom_emb, ((0, 1), (0, 0))) @ W_emb[:EMB]
    A2 = jnp.pad(atom_emb, ((0, 1), (0, 0))) @ W_emb[EMB:2 * EMB]
    W3p = jnp.pad(W_emb[2 * EMB:], ((0, 2), (0, 0)))
    Wsbf_bd = jnp.zeros((48, 56), f32)
    for blk in range(NBLK):
        for l in range(NS):
            Wsbf_bd = Wsbf_bd.at[l * NR:(l + 1) * NR,
                                 blk * 28 + l * NB:blk * 28 + (l + 1) * NB].set(
                Wi_sbf[blk, l * NR:(l + 1) * NR, :])
    Wo_rbf8 = jnp.pad(Wo_rbf, ((0, 0), (0, 2), (0, 0)))
    Wi_rbf8 = jnp.pad(Wi_rbf, ((0, 0), (0, 2), (0, 0)))

    pos_z = jnp.concatenate(
        [pos, Z.astype(f32)[:, None], jnp.zeros((N, 124), f32)], axis=1)
    EP = 163840
    padi = jnp.arange(EP - E, dtype=jnp.int32) % N
    src_p = jnp.concatenate([src.astype(jnp.int32), padi])
    dst_p = jnp.concatenate([dst.astype(jnp.int32), padi])
    ts_i = ts.astype(jnp.int32)
    td_i = td.astype(jnp.int32)
    dst3 = dst.astype(jnp.int32).reshape(2, 16, 125, 40)
    zrows = jnp.zeros((N // 2, EMB), f32)

    ps = _sc_gather(pos_z, src_p)[:E]
    pd = _sc_gather(pos_z, dst_p)[:E]

    m, rbf8, geo, t0 = _edge_stage(ps, pd, A1, A2, W3p, Wsbf_bd, Wo_rbf8[0])

    geo_t = _sc_gather(geo, ts_i)
    vec_td = _sc_gather(geo, td_i)
    sbf8 = _tri_init_stage(geo_t, vec_td)

    node0 = _sc_scatter_node(t0, dst3, zrows, N, 40)
    P = _node_stage(node0, Wo_dense[0], Wo_out[0])

    for i in range(NBLK):
        x = _xsrc_stage(m, Wi_src[i])
        xk = _sc_gather(x, ts_i)
        Bcat = Wi_bil[i].reshape(EMB, NB * EMB).astype(jnp.bfloat16)
        msg = _bilinear_stage(xk, sbf8, Bcat, i)
        agg = jax.ops.segment_sum(msg, td, num_segments=E)
        m, t = _post_stage(m, rbf8, agg, Wi_rbf8[i], Wi_before[i],
                           Wi_after[i, 0], Wi_after[i, 1], Wo_rbf8[i + 1])
        node = _sc_scatter_node(t, dst3, zrows, N, 40)
        P = P + _node_stage(node, Wo_dense[i + 1], Wo_out[i + 1])

    return P
